# baseline (speedup 1.0000x reference)
import jax
import jax.numpy as jnp
from jax.experimental import pallas as pl
from jax.experimental.pallas import tpu as pltpu

_ALPHA_LN = 2.302585092994046

_CB = 64
_VB = 4
_SCHUNK = 512
_P = 64
_K = 8

_SORT8_NET = (
    (0, 1), (2, 3), (4, 5), (6, 7),
    (0, 2), (1, 3), (4, 6), (5, 7),
    (1, 2), (5, 6),
    (0, 4), (1, 5), (2, 6), (3, 7),
    (2, 4), (3, 5),
    (1, 2), (3, 4), (5, 6),
)


def _ce(lst, i, j):
    a, b = lst[i], lst[j]
    lst[i] = jnp.minimum(a, b)
    lst[j] = jnp.maximum(a, b)


def _sort8(vals):
    lst = list(vals)
    for i, j in _SORT8_NET:
        _ce(lst, i, j)
    return lst


def _merge_bottom8(A, B):
    C = [jnp.minimum(A[i], B[7 - i]) for i in range(8)]
    for d in (4, 2, 1):
        for i in range(8):
            if (i & d) == 0 and (i | d) < 8:
                _ce(C, i, i + d)
    return C


def _body(c_ref, inv_ref, x_ref, o_ref):
    for vv in range(_VB):
        xb = x_ref[vv]
        R = None
        for g in range(_P // 8):
            grp = []
            for t in range(8):
                p = g * 8 + t
                d = xb - c_ref[vv, 0, p]
                grp.append((d * d) * inv_ref[vv, 0, p])
            grp = _sort8(grp)
            R = grp if R is None else _merge_bottom8(R, grp)
        for j in range(_K):
            u = R[j].astype(jnp.float32)
            o_ref[:, vv, j, :] = jnp.where(u <= _ALPHA_LN, jnp.exp(-u), 0.0)


def kernel(x, fuzzy_sets, k):
    B, V, S = x.shape
    c = fuzzy_sets[:, :, 0].reshape(V, 1, _P).astype(jnp.bfloat16)
    sig = fuzzy_sets[:, :, 1]
    inv = (1.0 / (2.0 * sig * sig)).reshape(V, 1, _P).astype(jnp.bfloat16)
    grid = (B // _CB, V // _VB, S // _SCHUNK)
    out = pl.pallas_call(
        _body,
        grid=grid,
        in_specs=[
            pl.BlockSpec((_VB, 1, _P), lambda i, j, s: (j, 0, 0), memory_space=pltpu.SMEM),
            pl.BlockSpec((_VB, 1, _P), lambda i, j, s: (j, 0, 0), memory_space=pltpu.SMEM),
            pl.BlockSpec((_VB, _CB, _SCHUNK), lambda i, j, s: (j, i, s)),
        ],
        out_specs=pl.BlockSpec((_CB, _VB, _K, _SCHUNK), lambda i, j, s: (i, j, 0, s)),
        out_shape=jax.ShapeDtypeStruct((B, V, _K, S), jnp.float32),
        compiler_params=pltpu.CompilerParams(
            dimension_semantics=("parallel", "parallel", "parallel")),
    )(c, inv, jnp.transpose(x, (1, 0, 2)).astype(jnp.bfloat16))
    return jnp.transpose(out, (0, 1, 3, 2))


# speedup vs baseline: 1.0110x; 1.0110x over previous
import jax
import jax.numpy as jnp
from jax.experimental import pallas as pl
from jax.experimental.pallas import tpu as pltpu

_ALPHA_LN = 2.302585092994046

_CB = 64
_VB = 2
_SCHUNK = 512
_P = 64
_K = 8

_SORT8_NET = (
    (0, 1), (2, 3), (4, 5), (6, 7),
    (0, 2), (1, 3), (4, 6), (5, 7),
    (1, 2), (5, 6),
    (0, 4), (1, 5), (2, 6), (3, 7),
    (2, 4), (3, 5),
    (1, 2), (3, 4), (5, 6),
)


def _ce(lst, i, j):
    a, b = lst[i], lst[j]
    lst[i] = jnp.minimum(a, b)
    lst[j] = jnp.maximum(a, b)


def _sort8(vals):
    lst = list(vals)
    for i, j in _SORT8_NET:
        _ce(lst, i, j)
    return lst


def _merge_bottom8(A, B):
    C = [jnp.minimum(A[i], B[7 - i]) for i in range(8)]
    for d in (4, 2, 1):
        for i in range(8):
            if (i & d) == 0 and (i | d) < 8:
                _ce(C, i, i + d)
    return C


def _body(c_ref, inv_ref, x_ref, o_ref):
    for vv in range(_VB):
        xb = x_ref[vv]
        R = None
        for g in range(_P // 8):
            grp = []
            for t in range(8):
                p = g * 8 + t
                d = xb - c_ref[vv, 0, p]
                grp.append((d * d) * inv_ref[vv, 0, p])
            grp = _sort8(grp)
            R = grp if R is None else _merge_bottom8(R, grp)
        for j in range(_K):
            u = R[j].astype(jnp.float32)
            o_ref[:, vv, j, :] = jnp.where(u <= _ALPHA_LN, jnp.exp(-u), 0.0)


def kernel(x, fuzzy_sets, k):
    B, V, S = x.shape
    c = fuzzy_sets[:, :, 0].reshape(V, 1, _P).astype(jnp.bfloat16)
    sig = fuzzy_sets[:, :, 1]
    inv = (1.0 / (2.0 * sig * sig)).reshape(V, 1, _P).astype(jnp.bfloat16)
    grid = (B // _CB, V // _VB, S // _SCHUNK)
    out = pl.pallas_call(
        _body,
        grid=grid,
        in_specs=[
            pl.BlockSpec((_VB, 1, _P), lambda i, j, s: (j, 0, 0), memory_space=pltpu.SMEM),
            pl.BlockSpec((_VB, 1, _P), lambda i, j, s: (j, 0, 0), memory_space=pltpu.SMEM),
            pl.BlockSpec((_VB, _CB, _SCHUNK), lambda i, j, s: (j, i, s)),
        ],
        out_specs=pl.BlockSpec((_CB, _VB, _K, _SCHUNK), lambda i, j, s: (i, j, 0, s)),
        out_shape=jax.ShapeDtypeStruct((B, V, _K, S), jnp.float32),
        compiler_params=pltpu.CompilerParams(
            dimension_semantics=("parallel", "parallel", "parallel")),
    )(c, inv, jnp.transpose(x, (1, 0, 2)).astype(jnp.bfloat16))
    return jnp.transpose(out, (0, 1, 3, 2))
